# Initial kernel scaffold; baseline (speedup 1.0000x reference)
import jax
import jax.numpy as jnp
from jax.experimental import pallas as pl


def kernel(x, W_enc, b_enc, W_dec, b_dec):
    raise NotImplementedError("write your pallas kernel here")


# speedup vs baseline: 11.4259x; 11.4259x over previous
import functools

import jax
import jax.numpy as jnp
from jax.experimental import pallas as pl
from jax.experimental.pallas import tpu as pltpu

_K = 64


def _enc_body(x_ref, w_ref, benc_ref, bdec_ref, pre_ref):
    xc = x_ref[...] - bdec_ref[...]
    acc = jax.lax.dot_general(
        xc, w_ref[...],
        dimension_numbers=(((1,), (1,)), ((), ())),
        preferred_element_type=jnp.float32)
    pre_ref[...] = acc + benc_ref[...]


def _encode(x, w_enc, benc2, bdec2, br, bs):
    n_tok, d_in = x.shape
    d_sae = w_enc.shape[0]
    grid = (n_tok // br, d_sae // bs)
    return pl.pallas_call(
        _enc_body,
        grid=grid,
        in_specs=[
            pl.BlockSpec((br, d_in), lambda t, s: (t, 0)),
            pl.BlockSpec((bs, d_in), lambda t, s: (s, 0)),
            pl.BlockSpec((1, bs), lambda t, s: (0, s)),
            pl.BlockSpec((1, d_in), lambda t, s: (0, 0)),
        ],
        out_specs=pl.BlockSpec((br, bs), lambda t, s: (t, s)),
        out_shape=jax.ShapeDtypeStruct((n_tok, d_sae), jnp.float32),
        compiler_params=pltpu.CompilerParams(
            dimension_semantics=("parallel", "parallel")),
    )(x, w_enc, benc2, bdec2)


def _sel_body(pre_ref, acts_ref, key_ref):
    pre = pre_ref[...]
    kbits = jax.lax.bitcast_convert_type(pre, jnp.int32)
    key = jnp.where(kbits < 0, kbits ^ jnp.int32(0x7FFFFFFF), kbits)
    key_ref[...] = key

    cnt_pos = jnp.sum((key >= 0).astype(jnp.int32), axis=1, keepdims=True)
    minint = jnp.int32(-2147483648)
    base = jnp.where(cnt_pos >= _K, jnp.int32(0), minint)

    def body(i, base):
        bit = jnp.int32(1) << (jnp.int32(30) - i)
        cand = base + bit
        cnt = jnp.sum((key_ref[...] >= cand).astype(jnp.int32),
                      axis=1, keepdims=True)
        return jnp.where(cnt >= _K, cand, base)

    thr = jax.lax.fori_loop(0, 31, body, base)
    keyv = key_ref[...]
    acts_ref[...] = jnp.where((keyv >= thr) & (pre > 0.0), pre, 0.0)


def _select(pre, br):
    n_tok, d_sae = pre.shape
    return pl.pallas_call(
        _sel_body,
        grid=(n_tok // br,),
        in_specs=[pl.BlockSpec((br, d_sae), lambda t: (t, 0))],
        out_specs=pl.BlockSpec((br, d_sae), lambda t: (t, 0)),
        out_shape=jax.ShapeDtypeStruct((n_tok, d_sae), jnp.float32),
        scratch_shapes=[pltpu.VMEM((br, d_sae), jnp.int32)],
        compiler_params=pltpu.CompilerParams(
            dimension_semantics=("parallel",)),
    )(pre)


def _dec_body(acts_ref, w_ref, bdec_ref, out_ref):
    k = pl.program_id(1)
    a = acts_ref[...].astype(jnp.bfloat16)
    part = jax.lax.dot_general(
        a, w_ref[...],
        dimension_numbers=(((1,), (1,)), ((), ())),
        preferred_element_type=jnp.float32)

    @pl.when(k == 0)
    def _init():
        out_ref[...] = part + bdec_ref[...]

    @pl.when(k > 0)
    def _acc():
        out_ref[...] += part


def _decode(acts, wdec_t_bf, bdec2, br, bk):
    n_tok, d_sae = acts.shape
    d_in = wdec_t_bf.shape[0]
    grid = (n_tok // br, d_sae // bk)
    return pl.pallas_call(
        _dec_body,
        grid=grid,
        in_specs=[
            pl.BlockSpec((br, bk), lambda t, k: (t, k)),
            pl.BlockSpec((d_in, bk), lambda t, k: (0, k)),
            pl.BlockSpec((1, d_in), lambda t, k: (0, 0)),
        ],
        out_specs=pl.BlockSpec((br, d_in), lambda t, k: (t, 0)),
        out_shape=jax.ShapeDtypeStruct((n_tok, d_in), jnp.float32),
        compiler_params=pltpu.CompilerParams(
            dimension_semantics=("parallel", "arbitrary")),
    )(acts, wdec_t_bf, bdec2)


def kernel(x, W_enc, b_enc, W_dec, b_dec):
    n_tok, d_in = x.shape
    d_sae = W_enc.shape[0]
    benc2 = b_enc.reshape(1, d_sae)
    bdec2 = b_dec.reshape(1, d_in)

    br1 = min(1024, n_tok)
    bs1 = min(1024, d_sae)
    pre = _encode(x, W_enc, benc2, bdec2, br1, bs1)

    br2 = min(128, n_tok)
    acts = _select(pre, br2)

    wdec_bf = W_dec.astype(jnp.bfloat16)
    br3 = min(1024, n_tok)
    bk3 = min(2048, d_sae)
    recon = _decode(acts, wdec_bf, bdec2, br3, bk3)
    return (recon, acts)


# speedup vs baseline: 12.7019x; 1.1117x over previous
import functools

import jax
import jax.numpy as jnp
from jax.experimental import pallas as pl
from jax.experimental.pallas import tpu as pltpu

_K = 64


def _enc_body(x_ref, w_ref, benc_ref, bdec_ref, pre_ref):
    xc = x_ref[...] - bdec_ref[...]
    acc = jax.lax.dot_general(
        xc, w_ref[...],
        dimension_numbers=(((1,), (1,)), ((), ())),
        preferred_element_type=jnp.float32)
    pre_ref[...] = acc + benc_ref[...]


def _encode(x, w_enc, benc2, bdec2, br, bs):
    n_tok, d_in = x.shape
    d_sae = w_enc.shape[0]
    grid = (n_tok // br, d_sae // bs)
    return pl.pallas_call(
        _enc_body,
        grid=grid,
        in_specs=[
            pl.BlockSpec((br, d_in), lambda t, s: (t, 0)),
            pl.BlockSpec((bs, d_in), lambda t, s: (s, 0)),
            pl.BlockSpec((1, bs), lambda t, s: (0, s)),
            pl.BlockSpec((1, d_in), lambda t, s: (0, 0)),
        ],
        out_specs=pl.BlockSpec((br, bs), lambda t, s: (t, s)),
        out_shape=jax.ShapeDtypeStruct((n_tok, d_sae), jnp.float32),
        compiler_params=pltpu.CompilerParams(
            dimension_semantics=("parallel", "parallel")),
    )(x, w_enc, benc2, bdec2)


def _sel_body(pre_ref, acts_ref, key_ref):
    pre = pre_ref[...]
    kbits = jax.lax.bitcast_convert_type(pre, jnp.int32)
    key = jnp.where(kbits < 0, kbits ^ jnp.int32(0x7FFFFFFF), kbits)
    key_ref[...] = key

    cnt_pos = jnp.sum((key >= 0).astype(jnp.int32), axis=1, keepdims=True)
    minint = jnp.int32(-2147483648)
    base0 = jnp.where(cnt_pos >= _K, jnp.int32(0), minint)
    found0 = jnp.zeros_like(base0)

    def cond(carry):
        b, base, thr, found = carry
        return (b >= 0) & (jnp.min(found) == 0)

    def body(carry):
        b, base, thr, found = carry
        cand = base + (jnp.int32(1) << b)
        cnt = jnp.sum((key_ref[...] >= cand).astype(jnp.int32),
                      axis=1, keepdims=True)
        hit = ((cnt == _K) & (found == 0)).astype(jnp.int32)
        thr = jnp.where(hit == 1, cand, thr)
        found = found | hit
        base = jnp.where(cnt >= _K, cand, base)
        return (b - jnp.int32(1), base, thr, found)

    _, base, thr, found = jax.lax.while_loop(
        cond, body, (jnp.int32(30), base0, base0, found0))
    thr = jnp.where(found == 1, thr, base)
    keyv = key_ref[...]
    acts_ref[...] = jnp.where((keyv >= thr) & (pre > 0.0), pre, 0.0)


def _select(pre, br):
    n_tok, d_sae = pre.shape
    return pl.pallas_call(
        _sel_body,
        grid=(n_tok // br,),
        in_specs=[pl.BlockSpec((br, d_sae), lambda t: (t, 0))],
        out_specs=pl.BlockSpec((br, d_sae), lambda t: (t, 0)),
        out_shape=jax.ShapeDtypeStruct((n_tok, d_sae), jnp.float32),
        scratch_shapes=[pltpu.VMEM((br, d_sae), jnp.int32)],
        compiler_params=pltpu.CompilerParams(
            dimension_semantics=("parallel",)),
    )(pre)


def _dec_body(acts_ref, w_ref, bdec_ref, out_ref):
    k = pl.program_id(1)
    a = acts_ref[...].astype(jnp.bfloat16)
    part = jax.lax.dot_general(
        a, w_ref[...],
        dimension_numbers=(((1,), (1,)), ((), ())),
        preferred_element_type=jnp.float32)

    @pl.when(k == 0)
    def _init():
        out_ref[...] = part + bdec_ref[...]

    @pl.when(k > 0)
    def _acc():
        out_ref[...] += part


def _decode(acts, wdec_t_bf, bdec2, br, bk):
    n_tok, d_sae = acts.shape
    d_in = wdec_t_bf.shape[0]
    grid = (n_tok // br, d_sae // bk)
    return pl.pallas_call(
        _dec_body,
        grid=grid,
        in_specs=[
            pl.BlockSpec((br, bk), lambda t, k: (t, k)),
            pl.BlockSpec((d_in, bk), lambda t, k: (0, k)),
            pl.BlockSpec((1, d_in), lambda t, k: (0, 0)),
        ],
        out_specs=pl.BlockSpec((br, d_in), lambda t, k: (t, 0)),
        out_shape=jax.ShapeDtypeStruct((n_tok, d_in), jnp.float32),
        compiler_params=pltpu.CompilerParams(
            dimension_semantics=("parallel", "arbitrary")),
    )(acts, wdec_t_bf, bdec2)


def kernel(x, W_enc, b_enc, W_dec, b_dec):
    n_tok, d_in = x.shape
    d_sae = W_enc.shape[0]
    benc2 = b_enc.reshape(1, d_sae)
    bdec2 = b_dec.reshape(1, d_in)

    br1 = min(1024, n_tok)
    bs1 = min(1024, d_sae)
    pre = _encode(x, W_enc, benc2, bdec2, br1, bs1)

    br2 = min(128, n_tok)
    acts = _select(pre, br2)

    wdec_bf = W_dec.astype(jnp.bfloat16)
    br3 = min(1024, n_tok)
    bk3 = min(2048, d_sae)
    recon = _decode(acts, wdec_bf, bdec2, br3, bk3)
    return (recon, acts)
